# baseline (speedup 1.0000x reference)
import functools

import jax
import jax.numpy as jnp
from jax import lax
from jax.experimental import pallas as pl
from jax.experimental.pallas import tpu as pltpu
from jax.experimental.pallas import tpu_sc as plsc

N = 10000
D = 128
E = 320000

_NUM_CORES = 2
_NUM_SUBCORES = 16
_NW = _NUM_CORES * _NUM_SUBCORES
_EPW = E // _NW
_L = 16
_DN = (((1,), (1,)), ((), ()))


_XBLK = 1024


def _tc_project(x_ref, wun_ref, wp_ref, bun_ref, bp_ref,
                un_ref, p0_ref, p1_ref):
    x = x_ref[...]
    un_ref[:, pl.ds(0, N)] = lax.dot_general(
        wun_ref[...], x, _DN, preferred_element_type=jnp.float32
    ) + bun_ref[0, 0]
    p0_ref[:, pl.ds(0, N)] = lax.dot_general(
        wp_ref[:, pl.ds(0, D)], x, _DN, preferred_element_type=jnp.float32)
    p1_ref[:, pl.ds(0, N)] = lax.dot_general(
        wp_ref[:, pl.ds(D, D)], x, _DN, preferred_element_type=jnp.float32
    ) + bp_ref[0, 0]


_WLEN = _EPW + 112
_NITER = _WLEN // _L
_NPAD = _WLEN
_SPLIT = 5120


def _sc_kernel(un_hbm, p0_hbm, p1_hbm, src_hbm, dst_hbm, out_hbm,
               src_v, dst_v, p0_v, p1_v, out_v, un_v,
               sem0, sem1, sem2, sem3, sem4, sem5):
    wid = lax.axis_index("s") * _NUM_CORES + lax.axis_index("c")
    delta = (16 * (wid + 1)) % 128
    e0 = pl.multiple_of(
        jnp.where(wid == 0, 0,
                  jnp.where(wid == _NW - 1, E - _WLEN, wid * _EPW - delta)),
        16)
    cp_p0 = pltpu.make_async_copy(p0_hbm.at[0, pl.ds(0, _NPAD)], p0_v, sem2)
    cp_p1 = pltpu.make_async_copy(p1_hbm.at[0, pl.ds(0, _NPAD)], p1_v, sem3)
    cp_p0.start()
    cp_p1.start()
    cp_src_a = pltpu.make_async_copy(
        src_hbm.at[pl.ds(e0, _SPLIT)], src_v.at[pl.ds(0, _SPLIT)], sem0)
    cp_dst_a = pltpu.make_async_copy(
        dst_hbm.at[pl.ds(e0, _SPLIT)], dst_v.at[pl.ds(0, _SPLIT)], sem1)
    cp_src_b = pltpu.make_async_copy(
        src_hbm.at[pl.ds(e0 + _SPLIT, _WLEN - _SPLIT)],
        src_v.at[pl.ds(_SPLIT, _WLEN - _SPLIT)], sem4)
    cp_dst_b = pltpu.make_async_copy(
        dst_hbm.at[pl.ds(e0 + _SPLIT, _WLEN - _SPLIT)],
        dst_v.at[pl.ds(_SPLIT, _WLEN - _SPLIT)], sem5)
    cp_src_a.start()
    cp_dst_a.start()
    cp_src_b.start()
    cp_dst_b.start()

    @pl.when(jnp.logical_and(wid >= 1, wid <= 9))
    def _copy_unary_main():
        off = pl.multiple_of((wid - 1) * 1024, 128)
        pltpu.sync_copy(un_hbm.at[0, pl.ds(off, 1024)], un_v)
        pltpu.sync_copy(un_v, out_hbm.at[0, pl.ds(off, 1024)])

    @pl.when(wid == 10)
    def _copy_unary_tail():
        pltpu.sync_copy(un_hbm.at[0, pl.ds(9216, 768)], un_v.at[pl.ds(0, 768)])
        pltpu.sync_copy(un_v.at[pl.ds(0, 768)], out_hbm.at[0, pl.ds(9216, 768)])

    @pl.when(wid == 0)
    def _copy_unary_head():
        pltpu.sync_copy(un_hbm.at[0, pl.ds(N - _L, 128)],
                        out_v.at[pl.ds(0, 128)])

    boff = jnp.where(wid == 0, _L, 0)

    cp_src_a.wait()
    cp_dst_a.wait()
    cp_p0.wait()
    cp_p1.wait()

    @plsc.parallel_loop(0, _SPLIT // _L, unroll=4)
    def _body_a(i):
        off = i * _L
        s = src_v[pl.ds(off, _L)]
        d = dst_v[pl.ds(off, _L)]
        g0 = plsc.load_gather(p0_v, [s])
        g1 = plsc.load_gather(p1_v, [d])
        out_v[pl.ds(boff + off, _L)] = g0 + g1

    cp_src_b.wait()
    cp_dst_b.wait()

    @plsc.parallel_loop(_SPLIT // _L, _NITER, unroll=4)
    def _body_b(i):
        off = i * _L
        s = src_v[pl.ds(off, _L)]
        d = dst_v[pl.ds(off, _L)]
        g0 = plsc.load_gather(p0_v, [s])
        g1 = plsc.load_gather(p1_v, [d])
        out_v[pl.ds(boff + off, _L)] = g0 + g1

    a0 = pl.multiple_of(N + wid * _EPW - delta, 128)

    @pl.when(wid < _NW - 1)
    def _store_full():
        pltpu.sync_copy(out_v.at[pl.ds(0, _WLEN)],
                        out_hbm.at[0, pl.ds(a0, _WLEN)])

    @pl.when(wid == _NW - 1)
    def _store_last():
        pltpu.sync_copy(out_v.at[pl.ds(112, _EPW - _L)],
                        out_hbm.at[0, pl.ds(a0, _EPW - _L)])
        pltpu.sync_copy(out_v.at[pl.ds(_WLEN - _L, _L)],
                        out_hbm.at[0, pl.ds(N + E - _L, _L)])


def kernel(input_states, W_un, b_un, W_pair, b_pair, pair_src, pair_dst):
    un, p0, p1 = pl.pallas_call(
        _tc_project,
        out_shape=[
            jax.ShapeDtypeStruct((1, _NPAD), jnp.float32),
            jax.ShapeDtypeStruct((1, _NPAD), jnp.float32),
            jax.ShapeDtypeStruct((1, _NPAD), jnp.float32),
        ],
    )(input_states, W_un.reshape(1, D), W_pair.reshape(1, 2 * D),
      b_un.reshape(1, 1), b_pair.reshape(1, 1))

    mesh = plsc.VectorSubcoreMesh(core_axis_name="c", subcore_axis_name="s")
    out = pl.kernel(
        _sc_kernel,
        mesh=mesh,
        compiler_params=pltpu.CompilerParams(needs_layout_passes=False),
        out_type=jax.ShapeDtypeStruct((1, N + E), jnp.float32),
        scratch_types=[
            pltpu.VMEM((_WLEN,), jnp.int32),
            pltpu.VMEM((_WLEN,), jnp.int32),
            pltpu.VMEM((_NPAD,), jnp.float32),
            pltpu.VMEM((_NPAD,), jnp.float32),
            pltpu.VMEM((_WLEN + _L,), jnp.float32),
            pltpu.VMEM((1024,), jnp.float32),
            pltpu.SemaphoreType.DMA,
            pltpu.SemaphoreType.DMA,
            pltpu.SemaphoreType.DMA,
            pltpu.SemaphoreType.DMA,
            pltpu.SemaphoreType.DMA,
            pltpu.SemaphoreType.DMA,
        ],
    )(un, p0, p1, pair_src, pair_dst)

    return out.reshape(N + E, 1)


# speedup vs baseline: 1.0397x; 1.0397x over previous
import functools

import jax
import jax.numpy as jnp
from jax import lax
from jax.experimental import pallas as pl
from jax.experimental.pallas import tpu as pltpu
from jax.experimental.pallas import tpu_sc as plsc

N = 10000
D = 128
E = 320000

_NUM_CORES = 2
_NUM_SUBCORES = 16
_NW = _NUM_CORES * _NUM_SUBCORES
_EPW = E // _NW
_L = 16
_DN = (((1,), (1,)), ((), ()))


_XBLK = 1024


def _tc_project(x_ref, wun_ref, wp_ref, bun_ref, bp_ref,
                un_ref, p0_ref, p1_ref):
    x = x_ref[...]
    w3 = jnp.concatenate([wun_ref[...], wp_ref[...]], axis=0)
    out3 = lax.dot_general(w3, x, _DN,
                           preferred_element_type=jnp.float32)
    un_ref[:, pl.ds(0, N)] = out3[0:1, :] + bun_ref[0, 0]
    p0_ref[:, pl.ds(0, N)] = out3[1:2, :]
    p1_ref[:, pl.ds(0, N)] = out3[2:3, :] + bp_ref[0, 0]


_WLEN = _EPW + 112
_NITER = _WLEN // _L
_NPAD = _WLEN
_SPLIT = 5120


def _sc_kernel(un_hbm, p0_hbm, p1_hbm, src_hbm, dst_hbm, out_hbm,
               src_v, dst_v, p0_v, p1_v, out_v, un_v,
               sem0, sem1, sem2, sem3, sem4, sem5):
    wid = lax.axis_index("s") * _NUM_CORES + lax.axis_index("c")
    delta = (16 * (wid + 1)) % 128
    e0 = pl.multiple_of(
        jnp.where(wid == 0, 0,
                  jnp.where(wid == _NW - 1, E - _WLEN, wid * _EPW - delta)),
        16)
    cp_p0 = pltpu.make_async_copy(p0_hbm.at[0, pl.ds(0, _NPAD)], p0_v, sem2)
    cp_p1 = pltpu.make_async_copy(p1_hbm.at[0, pl.ds(0, _NPAD)], p1_v, sem3)
    cp_p0.start()
    cp_p1.start()
    cp_src_a = pltpu.make_async_copy(
        src_hbm.at[pl.ds(e0, _SPLIT)], src_v.at[pl.ds(0, _SPLIT)], sem0)
    cp_dst_a = pltpu.make_async_copy(
        dst_hbm.at[pl.ds(e0, _SPLIT)], dst_v.at[pl.ds(0, _SPLIT)], sem1)
    cp_src_b = pltpu.make_async_copy(
        src_hbm.at[pl.ds(e0 + _SPLIT, _WLEN - _SPLIT)],
        src_v.at[pl.ds(_SPLIT, _WLEN - _SPLIT)], sem4)
    cp_dst_b = pltpu.make_async_copy(
        dst_hbm.at[pl.ds(e0 + _SPLIT, _WLEN - _SPLIT)],
        dst_v.at[pl.ds(_SPLIT, _WLEN - _SPLIT)], sem5)
    cp_src_a.start()
    cp_dst_a.start()
    cp_src_b.start()
    cp_dst_b.start()

    @pl.when(jnp.logical_and(wid >= 1, wid <= 9))
    def _copy_unary_main():
        off = pl.multiple_of((wid - 1) * 1024, 128)
        pltpu.sync_copy(un_hbm.at[0, pl.ds(off, 1024)], un_v)
        pltpu.sync_copy(un_v, out_hbm.at[0, pl.ds(off, 1024)])

    @pl.when(wid == 10)
    def _copy_unary_tail():
        pltpu.sync_copy(un_hbm.at[0, pl.ds(9216, 768)], un_v.at[pl.ds(0, 768)])
        pltpu.sync_copy(un_v.at[pl.ds(0, 768)], out_hbm.at[0, pl.ds(9216, 768)])

    @pl.when(wid == 0)
    def _copy_unary_head():
        pltpu.sync_copy(un_hbm.at[0, pl.ds(N - _L, 128)],
                        out_v.at[pl.ds(0, 128)])

    boff = jnp.where(wid == 0, _L, 0)

    cp_src_a.wait()
    cp_dst_a.wait()
    cp_p0.wait()
    cp_p1.wait()

    @plsc.parallel_loop(0, _SPLIT // _L, unroll=4)
    def _body_a(i):
        off = i * _L
        s = src_v[pl.ds(off, _L)]
        d = dst_v[pl.ds(off, _L)]
        g0 = plsc.load_gather(p0_v, [s])
        g1 = plsc.load_gather(p1_v, [d])
        out_v[pl.ds(boff + off, _L)] = g0 + g1

    cp_src_b.wait()
    cp_dst_b.wait()

    @plsc.parallel_loop(_SPLIT // _L, _NITER, unroll=4)
    def _body_b(i):
        off = i * _L
        s = src_v[pl.ds(off, _L)]
        d = dst_v[pl.ds(off, _L)]
        g0 = plsc.load_gather(p0_v, [s])
        g1 = plsc.load_gather(p1_v, [d])
        out_v[pl.ds(boff + off, _L)] = g0 + g1

    a0 = pl.multiple_of(N + wid * _EPW - delta, 128)

    @pl.when(wid < _NW - 1)
    def _store_full():
        pltpu.sync_copy(out_v.at[pl.ds(0, _WLEN)],
                        out_hbm.at[0, pl.ds(a0, _WLEN)])

    @pl.when(wid == _NW - 1)
    def _store_last():
        pltpu.sync_copy(out_v.at[pl.ds(112, _EPW - _L)],
                        out_hbm.at[0, pl.ds(a0, _EPW - _L)])
        pltpu.sync_copy(out_v.at[pl.ds(_WLEN - _L, _L)],
                        out_hbm.at[0, pl.ds(N + E - _L, _L)])


def kernel(input_states, W_un, b_un, W_pair, b_pair, pair_src, pair_dst):
    un, p0, p1 = pl.pallas_call(
        _tc_project,
        out_shape=[
            jax.ShapeDtypeStruct((1, _NPAD), jnp.float32),
            jax.ShapeDtypeStruct((1, _NPAD), jnp.float32),
            jax.ShapeDtypeStruct((1, _NPAD), jnp.float32),
        ],
    )(input_states, W_un.reshape(1, D), W_pair.reshape(2, D),
      b_un.reshape(1, 1), b_pair.reshape(1, 1))

    mesh = plsc.VectorSubcoreMesh(core_axis_name="c", subcore_axis_name="s")
    out = pl.kernel(
        _sc_kernel,
        mesh=mesh,
        compiler_params=pltpu.CompilerParams(needs_layout_passes=False),
        out_type=jax.ShapeDtypeStruct((1, N + E), jnp.float32),
        scratch_types=[
            pltpu.VMEM((_WLEN,), jnp.int32),
            pltpu.VMEM((_WLEN,), jnp.int32),
            pltpu.VMEM((_NPAD,), jnp.float32),
            pltpu.VMEM((_NPAD,), jnp.float32),
            pltpu.VMEM((_WLEN + _L,), jnp.float32),
            pltpu.VMEM((1024,), jnp.float32),
            pltpu.SemaphoreType.DMA,
            pltpu.SemaphoreType.DMA,
            pltpu.SemaphoreType.DMA,
            pltpu.SemaphoreType.DMA,
            pltpu.SemaphoreType.DMA,
            pltpu.SemaphoreType.DMA,
        ],
    )(un, p0, p1, pair_src, pair_dst)

    return out.reshape(N + E, 1)


# speedup vs baseline: 1.0431x; 1.0032x over previous
import functools

import jax
import jax.numpy as jnp
from jax import lax
from jax.experimental import pallas as pl
from jax.experimental.pallas import tpu as pltpu
from jax.experimental.pallas import tpu_sc as plsc

N = 10000
D = 128
E = 320000

_NUM_CORES = 2
_NUM_SUBCORES = 16
_NW = _NUM_CORES * _NUM_SUBCORES
_EPW = E // _NW
_L = 16
_DN = (((1,), (1,)), ((), ()))


_XBLK = 1024


def _tc_project(x_ref, wun_ref, wp_ref, bun_ref, bp_ref,
                un_ref, p0_ref, p1_ref):
    x = x_ref[...]
    w3 = jnp.concatenate([wun_ref[...], wp_ref[...]], axis=0)
    out3 = lax.dot_general(w3, x, _DN,
                           preferred_element_type=jnp.float32)
    un_ref[:, pl.ds(0, N)] = out3[0:1, :] + bun_ref[0, 0]
    p0_ref[:, pl.ds(0, N)] = out3[1:2, :]
    p1_ref[:, pl.ds(0, N)] = out3[2:3, :] + bp_ref[0, 0]


_WLEN = _EPW + 112
_NITER = _WLEN // _L
_NPAD = _WLEN
_SPLIT = 5120


def _sc_kernel(un_hbm, p0_hbm, p1_hbm, src_hbm, dst_hbm, out_hbm,
               src_v, dst_v, p0_v, p1_v, out_v, un_v,
               sem0, sem1, sem2, sem3, sem4, sem5):
    wid = lax.axis_index("s") * _NUM_CORES + lax.axis_index("c")
    delta = (16 * (wid + 1)) % 128
    e0 = pl.multiple_of(
        jnp.where(wid == 0, 0,
                  jnp.where(wid == _NW - 1, E - _WLEN, wid * _EPW - delta)),
        16)
    cp_p0 = pltpu.make_async_copy(p0_hbm.at[0, pl.ds(0, _NPAD)], p0_v, sem2)
    cp_p1 = pltpu.make_async_copy(p1_hbm.at[0, pl.ds(0, _NPAD)], p1_v, sem3)
    cp_p0.start()
    cp_p1.start()
    cp_src_a = pltpu.make_async_copy(
        src_hbm.at[pl.ds(e0, _SPLIT)], src_v.at[pl.ds(0, _SPLIT)], sem0)
    cp_dst_a = pltpu.make_async_copy(
        dst_hbm.at[pl.ds(e0, _SPLIT)], dst_v.at[pl.ds(0, _SPLIT)], sem1)
    cp_src_b = pltpu.make_async_copy(
        src_hbm.at[pl.ds(e0 + _SPLIT, _WLEN - _SPLIT)],
        src_v.at[pl.ds(_SPLIT, _WLEN - _SPLIT)], sem4)
    cp_dst_b = pltpu.make_async_copy(
        dst_hbm.at[pl.ds(e0 + _SPLIT, _WLEN - _SPLIT)],
        dst_v.at[pl.ds(_SPLIT, _WLEN - _SPLIT)], sem5)
    cp_src_a.start()
    cp_dst_a.start()

    @pl.when(jnp.logical_and(wid >= 1, wid <= 9))
    def _copy_unary_main():
        off = pl.multiple_of((wid - 1) * 1024, 128)
        pltpu.sync_copy(un_hbm.at[0, pl.ds(off, 1024)], un_v)
        pltpu.sync_copy(un_v, out_hbm.at[0, pl.ds(off, 1024)])

    @pl.when(wid == 10)
    def _copy_unary_tail():
        pltpu.sync_copy(un_hbm.at[0, pl.ds(9216, 768)], un_v.at[pl.ds(0, 768)])
        pltpu.sync_copy(un_v.at[pl.ds(0, 768)], out_hbm.at[0, pl.ds(9216, 768)])

    @pl.when(wid == 0)
    def _copy_unary_head():
        pltpu.sync_copy(un_hbm.at[0, pl.ds(N - _L, 128)],
                        out_v.at[pl.ds(0, 128)])

    boff = jnp.where(wid == 0, _L, 0)

    cp_src_a.wait()
    cp_dst_a.wait()
    cp_p0.wait()
    cp_p1.wait()
    cp_src_b.start()
    cp_dst_b.start()

    @plsc.parallel_loop(0, _SPLIT // _L, unroll=4)
    def _body_a(i):
        off = i * _L
        s = src_v[pl.ds(off, _L)]
        d = dst_v[pl.ds(off, _L)]
        g0 = plsc.load_gather(p0_v, [s])
        g1 = plsc.load_gather(p1_v, [d])
        out_v[pl.ds(boff + off, _L)] = g0 + g1

    cp_src_b.wait()
    cp_dst_b.wait()

    @plsc.parallel_loop(_SPLIT // _L, _NITER, unroll=4)
    def _body_b(i):
        off = i * _L
        s = src_v[pl.ds(off, _L)]
        d = dst_v[pl.ds(off, _L)]
        g0 = plsc.load_gather(p0_v, [s])
        g1 = plsc.load_gather(p1_v, [d])
        out_v[pl.ds(boff + off, _L)] = g0 + g1

    a0 = pl.multiple_of(N + wid * _EPW - delta, 128)

    @pl.when(wid < _NW - 1)
    def _store_full():
        pltpu.sync_copy(out_v.at[pl.ds(0, _WLEN)],
                        out_hbm.at[0, pl.ds(a0, _WLEN)])

    @pl.when(wid == _NW - 1)
    def _store_last():
        pltpu.sync_copy(out_v.at[pl.ds(112, _EPW - _L)],
                        out_hbm.at[0, pl.ds(a0, _EPW - _L)])
        pltpu.sync_copy(out_v.at[pl.ds(_WLEN - _L, _L)],
                        out_hbm.at[0, pl.ds(N + E - _L, _L)])


def kernel(input_states, W_un, b_un, W_pair, b_pair, pair_src, pair_dst):
    un, p0, p1 = pl.pallas_call(
        _tc_project,
        out_shape=[
            jax.ShapeDtypeStruct((1, _NPAD), jnp.float32),
            jax.ShapeDtypeStruct((1, _NPAD), jnp.float32),
            jax.ShapeDtypeStruct((1, _NPAD), jnp.float32),
        ],
    )(input_states, W_un.reshape(1, D), W_pair.reshape(2, D),
      b_un.reshape(1, 1), b_pair.reshape(1, 1))

    mesh = plsc.VectorSubcoreMesh(core_axis_name="c", subcore_axis_name="s")
    out = pl.kernel(
        _sc_kernel,
        mesh=mesh,
        compiler_params=pltpu.CompilerParams(needs_layout_passes=False),
        out_type=jax.ShapeDtypeStruct((1, N + E), jnp.float32),
        scratch_types=[
            pltpu.VMEM((_WLEN,), jnp.int32),
            pltpu.VMEM((_WLEN,), jnp.int32),
            pltpu.VMEM((_NPAD,), jnp.float32),
            pltpu.VMEM((_NPAD,), jnp.float32),
            pltpu.VMEM((_WLEN + _L,), jnp.float32),
            pltpu.VMEM((1024,), jnp.float32),
            pltpu.SemaphoreType.DMA,
            pltpu.SemaphoreType.DMA,
            pltpu.SemaphoreType.DMA,
            pltpu.SemaphoreType.DMA,
            pltpu.SemaphoreType.DMA,
            pltpu.SemaphoreType.DMA,
        ],
    )(un, p0, p1, pair_src, pair_dst)

    return out.reshape(N + E, 1)


# speedup vs baseline: 1.0573x; 1.0136x over previous
import functools

import jax
import jax.numpy as jnp
from jax import lax
from jax.experimental import pallas as pl
from jax.experimental.pallas import tpu as pltpu
from jax.experimental.pallas import tpu_sc as plsc

N = 10000
D = 128
E = 320000

_NUM_CORES = 2
_NUM_SUBCORES = 16
_NW = _NUM_CORES * _NUM_SUBCORES
_EPW = E // _NW
_L = 16
_DN = (((1,), (1,)), ((), ()))


_XBLK = 5120


def _tc_project(x_ref, wun_ref, wp_ref, bun_ref, bp_ref,
                un_ref, p0_ref, p1_ref):
    x = x_ref[...]
    w3 = jnp.concatenate([wun_ref[...], wp_ref[...]], axis=0)
    out3 = lax.dot_general(w3, x, _DN,
                           preferred_element_type=jnp.float32)
    un_ref[...] = out3[0:1, :] + bun_ref[0, 0]
    p0_ref[...] = out3[1:2, :]
    p1_ref[...] = out3[2:3, :] + bp_ref[0, 0]


_WLEN = _EPW + 112
_NITER = _WLEN // _L
_NPAD = _WLEN
_SPLIT = 5120


def _sc_kernel(un_hbm, p0_hbm, p1_hbm, src_hbm, dst_hbm, out_hbm,
               src_v, dst_v, p0_v, p1_v, out_v, un_v,
               sem0, sem1, sem2, sem3, sem4, sem5):
    wid = lax.axis_index("s") * _NUM_CORES + lax.axis_index("c")
    delta = (16 * (wid + 1)) % 128
    e0 = pl.multiple_of(
        jnp.where(wid == 0, 0,
                  jnp.where(wid == _NW - 1, E - _WLEN, wid * _EPW - delta)),
        16)
    cp_p0 = pltpu.make_async_copy(p0_hbm.at[0, pl.ds(0, _NPAD)], p0_v, sem2)
    cp_p1 = pltpu.make_async_copy(p1_hbm.at[0, pl.ds(0, _NPAD)], p1_v, sem3)
    cp_p0.start()
    cp_p1.start()
    cp_src_a = pltpu.make_async_copy(
        src_hbm.at[pl.ds(e0, _SPLIT)], src_v.at[pl.ds(0, _SPLIT)], sem0)
    cp_dst_a = pltpu.make_async_copy(
        dst_hbm.at[pl.ds(e0, _SPLIT)], dst_v.at[pl.ds(0, _SPLIT)], sem1)
    cp_src_b = pltpu.make_async_copy(
        src_hbm.at[pl.ds(e0 + _SPLIT, _WLEN - _SPLIT)],
        src_v.at[pl.ds(_SPLIT, _WLEN - _SPLIT)], sem4)
    cp_dst_b = pltpu.make_async_copy(
        dst_hbm.at[pl.ds(e0 + _SPLIT, _WLEN - _SPLIT)],
        dst_v.at[pl.ds(_SPLIT, _WLEN - _SPLIT)], sem5)
    cp_src_a.start()
    cp_dst_a.start()

    @pl.when(jnp.logical_and(wid >= 1, wid <= 9))
    def _copy_unary_main():
        off = pl.multiple_of((wid - 1) * 1024, 128)
        pltpu.sync_copy(un_hbm.at[0, pl.ds(off, 1024)], un_v)
        pltpu.sync_copy(un_v, out_hbm.at[0, pl.ds(off, 1024)])

    @pl.when(wid == 10)
    def _copy_unary_tail():
        pltpu.sync_copy(un_hbm.at[0, pl.ds(9216, 768)], un_v.at[pl.ds(0, 768)])
        pltpu.sync_copy(un_v.at[pl.ds(0, 768)], out_hbm.at[0, pl.ds(9216, 768)])

    @pl.when(wid == 0)
    def _copy_unary_head():
        pltpu.sync_copy(un_hbm.at[0, pl.ds(N - _L, 128)],
                        out_v.at[pl.ds(0, 128)])

    boff = jnp.where(wid == 0, _L, 0)

    cp_src_a.wait()
    cp_dst_a.wait()
    cp_p0.wait()
    cp_p1.wait()
    cp_src_b.start()
    cp_dst_b.start()

    @plsc.parallel_loop(0, _SPLIT // _L, unroll=4)
    def _body_a(i):
        off = i * _L
        s = src_v[pl.ds(off, _L)]
        d = dst_v[pl.ds(off, _L)]
        g0 = plsc.load_gather(p0_v, [s])
        g1 = plsc.load_gather(p1_v, [d])
        out_v[pl.ds(boff + off, _L)] = g0 + g1

    cp_src_b.wait()
    cp_dst_b.wait()

    @plsc.parallel_loop(_SPLIT // _L, _NITER, unroll=4)
    def _body_b(i):
        off = i * _L
        s = src_v[pl.ds(off, _L)]
        d = dst_v[pl.ds(off, _L)]
        g0 = plsc.load_gather(p0_v, [s])
        g1 = plsc.load_gather(p1_v, [d])
        out_v[pl.ds(boff + off, _L)] = g0 + g1

    a0 = pl.multiple_of(N + wid * _EPW - delta, 128)

    @pl.when(wid < _NW - 1)
    def _store_full():
        pltpu.sync_copy(out_v.at[pl.ds(0, _WLEN)],
                        out_hbm.at[0, pl.ds(a0, _WLEN)])

    @pl.when(wid == _NW - 1)
    def _store_last():
        pltpu.sync_copy(out_v.at[pl.ds(112, _EPW - _L)],
                        out_hbm.at[0, pl.ds(a0, _EPW - _L)])
        pltpu.sync_copy(out_v.at[pl.ds(_WLEN - _L, _L)],
                        out_hbm.at[0, pl.ds(N + E - _L, _L)])


def kernel(input_states, W_un, b_un, W_pair, b_pair, pair_src, pair_dst):
    _vec_spec = pl.BlockSpec((1, _XBLK), lambda i: (0, i))
    un, p0, p1 = pl.pallas_call(
        _tc_project,
        grid=(2,),
        in_specs=[
            pl.BlockSpec((_XBLK, D), lambda i: (i, 0)),
            pl.BlockSpec((1, D), lambda i: (0, 0)),
            pl.BlockSpec((2, D), lambda i: (0, 0)),
            pl.BlockSpec((1, 1), lambda i: (0, 0)),
            pl.BlockSpec((1, 1), lambda i: (0, 0)),
        ],
        out_specs=[_vec_spec, _vec_spec, _vec_spec],
        out_shape=[
            jax.ShapeDtypeStruct((1, _NPAD), jnp.float32),
            jax.ShapeDtypeStruct((1, _NPAD), jnp.float32),
            jax.ShapeDtypeStruct((1, _NPAD), jnp.float32),
        ],
    )(input_states, W_un.reshape(1, D), W_pair.reshape(2, D),
      b_un.reshape(1, 1), b_pair.reshape(1, 1))

    mesh = plsc.VectorSubcoreMesh(core_axis_name="c", subcore_axis_name="s")
    out = pl.kernel(
        _sc_kernel,
        mesh=mesh,
        compiler_params=pltpu.CompilerParams(needs_layout_passes=False),
        out_type=jax.ShapeDtypeStruct((1, N + E), jnp.float32),
        scratch_types=[
            pltpu.VMEM((_WLEN,), jnp.int32),
            pltpu.VMEM((_WLEN,), jnp.int32),
            pltpu.VMEM((_NPAD,), jnp.float32),
            pltpu.VMEM((_NPAD,), jnp.float32),
            pltpu.VMEM((_WLEN + _L,), jnp.float32),
            pltpu.VMEM((1024,), jnp.float32),
            pltpu.SemaphoreType.DMA,
            pltpu.SemaphoreType.DMA,
            pltpu.SemaphoreType.DMA,
            pltpu.SemaphoreType.DMA,
            pltpu.SemaphoreType.DMA,
            pltpu.SemaphoreType.DMA,
        ],
    )(un, p0, p1, pair_src, pair_dst)

    return out.reshape(N + E, 1)
